# baseline (speedup 1.0000x reference)
import jax
import jax.numpy as jnp
from jax import lax
from jax.experimental import pallas as pl
from jax.experimental.pallas import tpu as pltpu
from jax.experimental.pallas import tpu_sc as plsc

T, N, E, B, D, C = 4, 10000, 320000, 64, 128, 10
NS = 16
EPT = E // NS
K = 80
NCHUNK = EPT // K
NLIN = N // 10


def _sc_edge_kernel(src_hbm, dst_hbm, asrc_hbm, adst_hbm, ht_hbm,
                    eatt_hbm, d_hbm, msg_hbm, deg_hbm,
                    pn1_v, pn2_v,
                    sr0, sr1, dsc0, dsc1, dx0, dx1, six0, six1,
                    ec0, ec1, at0, at1, ro0, ro1,
                    ones_v, zline_v,
                    iss0, iss1, isd0, isd1, ise0, ise1,
                    gs0, gs1, ss0, ss1, ew0, ew1, os0, os1,
                    d_sh, deg_sh, msg_sh):
    cid = lax.axis_index("c")
    sid = lax.axis_index("s")
    ebase = sid * EPT
    SR = (sr0, sr1)
    DS = (dsc0, dsc1)
    DX = (dx0, dx1)
    SI = (six0, six1)
    EC = (ec0, ec1)
    AT = (at0, at1)
    RO = (ro0, ro1)
    ISS = (iss0, iss1)
    ISD = (isd0, isd1)
    ISE = (ise0, ise1)
    GS = (gs0, gs1)
    SS = (ss0, ss1)
    EW = (ew0, ew1)
    OS = (os0, os1)

    def eoff(g):
        return jnp.minimum(ebase + g * K, E - K)

    @pl.loop(0, K // 16)
    def _ones(i):
        ones_v[pl.ds(i * 16, 16)] = jnp.ones((16,), jnp.float32)

    @pl.loop(0, 1024 // 16)
    def _zl(i):
        zline_v[pl.ds(i * 16, 16)] = jnp.zeros((16,), jnp.float32)

    for tt in range(2):
        t = cid * 2 + tt
        tN = t * N
        tE = t * E

        @pl.loop(0, K)
        def _zr(r):
            for q in range(D // 16):
                ro0[r, pl.ds(q * 16, 16)] = jnp.zeros((16,), jnp.float32)

        @pl.when(sid < 10)
        def _zacc():
            pltpu.sync_copy(zline_v.at[pl.ds(0, NLIN)],
                            d_sh.at[pl.ds(sid * NLIN, NLIN)])
            for z in range(NLIN // K):
                pltpu.sync_copy(ro0, msg_sh.at[pl.ds(sid * NLIN + z * K, K)])
            pltpu.sync_copy(ro0.at[pl.ds(0, NLIN - (NLIN // K) * K)],
                            msg_sh.at[pl.ds(sid * NLIN + (NLIN // K) * K,
                                            NLIN - (NLIN // K) * K)])

        if tt == 0:
            @pl.when(sid < 10)
            def _zdeg():
                pltpu.sync_copy(zline_v.at[pl.ds(0, NLIN)],
                                deg_sh.at[pl.ds(sid * NLIN, NLIN)])

        pltpu.sync_copy(asrc_hbm.at[pl.ds(tN, N)], pn1_v)
        pltpu.sync_copy(adst_hbm.at[pl.ds(tN, N)], pn2_v)

        plsc.subcore_barrier()

        def ig1(g, b):
            pltpu.async_copy(src_hbm.at[pl.ds(eoff(g), K)], SR[b], ISS[b])
            pltpu.async_copy(dst_hbm.at[pl.ds(eoff(g), K)], DS[b], ISD[b])

        def g1(g, b):
            pltpu.make_async_copy(
                src_hbm.at[pl.ds(eoff(g), K)], SR[b], ISS[b]).wait()
            pltpu.make_async_copy(
                dst_hbm.at[pl.ds(eoff(g), K)], DS[b], ISD[b]).wait()
            for i in range(K // 16):
                sl = pl.ds(i * 16, 16)
                s16 = SR[b][sl]
                d16 = DS[b][sl]
                DX[b][sl] = d16
                av = plsc.load_gather(pn1_v, [s16])
                bv = plsc.load_gather(pn2_v, [d16])
                sc = av + bv
                sc = jnp.where(sc >= 0.0, sc, 0.2 * sc)
                EC[b][sl] = jnp.exp(sc)
            pltpu.async_copy(EC[b], d_sh.at[DX[b]], SS[b], add=True)
            if tt == 0:
                pltpu.async_copy(ones_v, deg_sh.at[DX[b]], OS[b], add=True)
            pltpu.async_copy(
                EC[b], eatt_hbm.at[pl.ds(tE + ebase + g * K, K)], EW[b])
            ig1(g + 2, b)

        def ws1(g, b):
            pltpu.make_async_copy(EC[b], d_sh.at[DX[b]], SS[b]).wait()
            if tt == 0:
                pltpu.make_async_copy(ones_v, deg_sh.at[DX[b]], OS[b]).wait()
            pltpu.make_async_copy(
                EC[b], eatt_hbm.at[pl.ds(tE + ebase + g * K, K)],
                EW[b]).wait()

        ig1(0, 0)
        ig1(1, 1)
        g1(0, 0)
        g1(1, 1)

        @pl.loop(0, NCHUNK // 2 - 1)
        def _p1(p):
            g = 2 * p
            ws1(g, 0)
            g1(g + 2, 0)
            ws1(g + 1, 1)
            g1(g + 3, 1)

        ws1(NCHUNK - 2, 0)
        ws1(NCHUNK - 1, 1)
        for b in range(2):
            pltpu.make_async_copy(
                src_hbm.at[pl.ds(eoff(NCHUNK + b), K)], SR[b], ISS[b]).wait()
            pltpu.make_async_copy(
                dst_hbm.at[pl.ds(eoff(NCHUNK + b), K)], DS[b], ISD[b]).wait()

        plsc.subcore_barrier()

        pltpu.sync_copy(d_sh, pn2_v)

        @pl.when(sid < 10)
        def _dexp():
            pltpu.sync_copy(pn2_v.at[pl.ds(sid * NLIN, NLIN)],
                            d_hbm.at[pl.ds(tN + sid * NLIN, NLIN)])

        if tt == 0:
            @pl.when(jnp.logical_and(cid == 0, sid < 10))
            def _gexp():
                pltpu.sync_copy(deg_sh.at[pl.ds(sid * NLIN, NLIN)],
                                pn1_v.at[pl.ds(0, NLIN)])
                pltpu.sync_copy(pn1_v.at[pl.ds(0, NLIN)],
                                deg_hbm.at[pl.ds(sid * NLIN, NLIN)])

        def ig2(g, b):
            pltpu.async_copy(src_hbm.at[pl.ds(eoff(g), K)], SR[b], ISS[b])
            pltpu.async_copy(dst_hbm.at[pl.ds(eoff(g), K)], DS[b], ISD[b])
            pltpu.async_copy(
                eatt_hbm.at[pl.ds(jnp.minimum(tE + ebase + g * K,
                                              tE + E - K), K)],
                EC[b], ISE[b])

        def g2(g, b):
            pltpu.make_async_copy(
                src_hbm.at[pl.ds(eoff(g), K)], SR[b], ISS[b]).wait()
            pltpu.make_async_copy(
                dst_hbm.at[pl.ds(eoff(g), K)], DS[b], ISD[b]).wait()
            pltpu.make_async_copy(
                eatt_hbm.at[pl.ds(tE + ebase + g * K, K)], EC[b],
                ISE[b]).wait()
            for i in range(K // 16):
                sl = pl.ds(i * 16, 16)
                s16 = SR[b][sl]
                d16 = DS[b][sl]
                SI[b][sl] = s16 + tN
                DX[b][sl] = d16
                dd = plsc.load_gather(pn2_v, [d16])
                AT[b][sl] = EC[b][sl] / (dd + 1e-9)
            pltpu.async_copy(ht_hbm.at[SI[b]], RO[b], GS[b])
            ig2(g + 2, b)

        def p2(g, b):
            pltpu.make_async_copy(ht_hbm.at[SI[b]], RO[b], GS[b]).wait()

            @pl.loop(0, K)
            def _scale(r):
                ab = plsc.load_gather(AT[b], [jnp.full((16,), r, jnp.int32)])
                for q in range(D // 16):
                    RO[b][r, pl.ds(q * 16, 16)] = (
                        RO[b][r, pl.ds(q * 16, 16)] * ab)

            pltpu.async_copy(RO[b], msg_sh.at[DX[b]], SS[b], add=True)
            pltpu.async_copy(
                AT[b], eatt_hbm.at[pl.ds(tE + ebase + g * K, K)], EW[b])

        def ws2(g, b):
            pltpu.make_async_copy(RO[b], msg_sh.at[DX[b]], SS[b]).wait()
            pltpu.make_async_copy(
                AT[b], eatt_hbm.at[pl.ds(tE + ebase + g * K, K)],
                EW[b]).wait()

        ig2(0, 0)
        ig2(1, 1)
        g2(0, 0)
        g2(1, 1)

        @pl.loop(0, NCHUNK // 2 - 1)
        def _p2(p):
            g = 2 * p
            p2(g, 0)
            p2(g + 1, 1)
            ws2(g, 0)
            g2(g + 2, 0)
            ws2(g + 1, 1)
            g2(g + 3, 1)

        p2(NCHUNK - 2, 0)
        p2(NCHUNK - 1, 1)
        ws2(NCHUNK - 2, 0)
        ws2(NCHUNK - 1, 1)
        for b in range(2):
            pltpu.make_async_copy(
                src_hbm.at[pl.ds(eoff(NCHUNK + b), K)], SR[b], ISS[b]).wait()
            pltpu.make_async_copy(
                dst_hbm.at[pl.ds(eoff(NCHUNK + b), K)], DS[b], ISD[b]).wait()
            pltpu.make_async_copy(
                eatt_hbm.at[pl.ds(jnp.minimum(tE + ebase + (NCHUNK + b) * K,
                                              tE + E - K), K)],
                EC[b], ISE[b]).wait()

        plsc.subcore_barrier()

        @pl.when(sid < 10)
        def _mexp():
            pltpu.sync_copy(msg_sh.at[pl.ds(sid * NLIN, NLIN)],
                            msg_hbm.at[pl.ds(tN + sid * NLIN, NLIN)])

        plsc.subcore_barrier()


def _run_sc_edge(asrc, adst, src, dst, ht):
    f32 = jnp.float32
    i32 = jnp.int32
    kern = pl.kernel(
        _sc_edge_kernel,
        out_type=[
            jax.ShapeDtypeStruct((T * E,), f32),
            jax.ShapeDtypeStruct((T * N,), f32),
            jax.ShapeDtypeStruct((T * N, D), f32),
            jax.ShapeDtypeStruct((N,), f32),
        ],
        mesh=plsc.VectorSubcoreMesh(core_axis_name="c", subcore_axis_name="s"),
        scratch_types=(
            [pltpu.VMEM((N,), f32) for _ in range(2)]
            + [pltpu.VMEM((K,), i32) for _ in range(8)]
            + [pltpu.VMEM((K,), f32) for _ in range(4)]
            + [pltpu.VMEM((K, D), f32) for _ in range(2)]
            + [pltpu.VMEM((K,), f32), pltpu.VMEM((1024,), f32)]
            + [pltpu.SemaphoreType.DMA for _ in range(14)]
            + [pltpu.VMEM_SHARED((N,), f32),
               pltpu.VMEM_SHARED((N,), f32),
               pltpu.VMEM_SHARED((N, D), f32)]
        ),
        compiler_params=pltpu.CompilerParams(needs_layout_passes=False),
    )
    return kern(src, dst, asrc, adst, ht)


RB = 1000
NB = N // RB


def _mm_kernel(x_ref, w_ref, o_ref):
    o_ref[...] = jnp.dot(x_ref[...], w_ref[...],
                         preferred_element_type=jnp.float32)


def _run_matmul(x_flat, w_aug):
    M = T * N
    return pl.pallas_call(
        _mm_kernel,
        grid=(M // RB,),
        in_specs=[
            pl.BlockSpec((RB, D), lambda i: (i, 0)),
            pl.BlockSpec((D, 2 * D), lambda i: (0, 0)),
        ],
        out_specs=pl.BlockSpec((RB, 2 * D), lambda i: (i, 0)),
        out_shape=jax.ShapeDtypeStruct((M, 2 * D), jnp.float32),
    )(x_flat, w_aug)


def _readout_kernel(msg_ref, n2g_ref, d_ref, deg_ref, g_ref, side_ref):
    j = pl.program_id(1)

    @pl.when(j == 0)
    def _init():
        g_ref[...] = jnp.zeros((1, B, D), jnp.float32)
        side_ref[...] = jnp.zeros((1, B, D), jnp.float32)

    nh = jnp.maximum(msg_ref[0], 0.0)
    n2g = n2g_ref[0, 0]
    gid = lax.broadcasted_iota(jnp.int32, (B, RB), 0)
    s_f = (jnp.broadcast_to(n2g[None, :], (B, RB)) == gid).astype(jnp.float32)

    dv = d_ref[0, 0]
    attsum = dv / (dv + 1e-9)
    degv = deg_ref[0, 0]
    lane = lax.broadcasted_iota(jnp.int32, (RB, D), 1)
    side = (jnp.where(lane == 0, attsum[:, None], 0.0)
            + jnp.where(lane == 1, degv[:, None], 0.0))

    g_ref[0] += jnp.dot(s_f, nh, preferred_element_type=jnp.float32)
    side_ref[0] += jnp.dot(s_f, side, preferred_element_type=jnp.float32)


def _run_readout(msg, n2g, d_arr, deg):
    msg_r = msg.reshape(T * NB, RB, D)
    n2g_r = n2g.reshape(NB, 1, RB)
    d_r = d_arr.reshape(T * NB, 1, RB)
    deg_r = deg.reshape(NB, 1, RB)
    return pl.pallas_call(
        _readout_kernel,
        grid=(T, NB),
        in_specs=[
            pl.BlockSpec((1, RB, D), lambda t, j: (t * NB + j, 0, 0)),
            pl.BlockSpec((1, 1, RB), lambda t, j: (j, 0, 0)),
            pl.BlockSpec((1, 1, RB), lambda t, j: (t * NB + j, 0, 0)),
            pl.BlockSpec((1, 1, RB), lambda t, j: (j, 0, 0)),
        ],
        out_specs=[
            pl.BlockSpec((1, B, D), lambda t, j: (t, 0, 0)),
            pl.BlockSpec((1, B, D), lambda t, j: (t, 0, 0)),
        ],
        out_shape=[
            jax.ShapeDtypeStruct((T, B, D), jnp.float32),
            jax.ShapeDtypeStruct((T, B, D), jnp.float32),
        ],
        compiler_params=pltpu.CompilerParams(
            dimension_semantics=("arbitrary", "arbitrary")),
    )(msg_r, n2g_r, d_r, deg_r)


def _lstm_kernel(g_ref, side_ref, wih_ref, whh_ref, wout_ref, bg_ref,
                 bo_ref, ps_ref, ra_ref):
    h = jnp.zeros((B, D), jnp.float32)
    c = jnp.zeros((B, D), jnp.float32)
    lane = lax.broadcasted_iota(jnp.int32, (B, D), 1)
    for t in range(T):
        xg = g_ref[t] + h
        gates = (jnp.dot(xg, wih_ref[...], preferred_element_type=jnp.float32)
                 + jnp.dot(h, whh_ref[...], preferred_element_type=jnp.float32)
                 + bg_ref[...])
        i_g = gates[:, 0 * D:1 * D]
        f_g = gates[:, 1 * D:2 * D]
        g_g = gates[:, 2 * D:3 * D]
        o_g = gates[:, 3 * D:4 * D]
        c = jax.nn.sigmoid(f_g) * c + jax.nn.sigmoid(i_g) * jnp.tanh(g_g)
        h = jax.nn.sigmoid(o_g) * jnp.tanh(c)
        logits = jnp.dot(h, wout_ref[...],
                         preferred_element_type=jnp.float32) + bo_ref[...]
        ml = jnp.where(lane < C, logits, -1e30)
        m = jnp.max(ml, axis=1, keepdims=True)
        e = jnp.exp(ml - m)
        ps_ref[t] = e / jnp.sum(e, axis=1, keepdims=True)

        sd = side_ref[t]
        ra = sd[:, 0:1] / (sd[:, 1:2] + 1e-9)
        ra_ref[t] = jnp.broadcast_to(ra, (B, D))


def _run_lstm(g_all, side_all, wih_t, whh_t, wout_t, bg, bo):
    return pl.pallas_call(
        _lstm_kernel,
        out_shape=[
            jax.ShapeDtypeStruct((T, B, D), jnp.float32),
            jax.ShapeDtypeStruct((T, B, D), jnp.float32),
        ],
    )(g_all, side_all, wih_t, whh_t, wout_t, bg, bo)


def kernel(x, edge_index, node2graph, mask, W, a_src, a_dst,
           W_ih, W_hh, b_ih, b_hh, W_out, b_out):
    f32 = jnp.float32
    x_flat = x.reshape(T * N, D).astype(f32)

    w_s = W @ a_src
    w_d = W @ a_dst
    w_aug = jnp.concatenate(
        [W, w_s[:, None], w_d[:, None], jnp.zeros((D, D - 2), f32)], axis=1)

    ht_aug = _run_matmul(x_flat, w_aug)
    ht = ht_aug[:, :D] + 0.0
    asrc = ht_aug[:, D] + 0.0
    adst = ht_aug[:, D + 1] + 0.0

    src = edge_index[0].astype(jnp.int32)
    dst = edge_index[1].astype(jnp.int32)

    eatt, d_arr, msg, deg = _run_sc_edge(asrc, adst, src, dst, ht)

    g_all, side_all = _run_readout(msg, node2graph.astype(jnp.int32),
                                   d_arr, deg)

    bg = (b_ih + b_hh).reshape(1, 4 * D).astype(f32)
    bo = jnp.concatenate([b_out, jnp.zeros((D - C,), f32)]).reshape(1, D)
    wout_t = jnp.concatenate([W_out.T, jnp.zeros((D, D - C), f32)], axis=1)

    ps_pad, ra_pad = _run_lstm(g_all, side_all,
                               W_ih.T.astype(f32), W_hh.T.astype(f32),
                               wout_t.astype(f32), bg, bo)

    ps = ps_pad[:, :, :C]
    rel_atts = ra_pad[:, :, 0]
    edge_atts = eatt.reshape(T, E)
    return (ps, rel_atts, edge_atts)


# speedup vs baseline: 1.0419x; 1.0419x over previous
import jax
import jax.numpy as jnp
from jax import lax
from jax.experimental import pallas as pl
from jax.experimental.pallas import tpu as pltpu
from jax.experimental.pallas import tpu_sc as plsc

T, N, E, B, D, C = 4, 10000, 320000, 64, 128, 10
NS = 16
EPT = E // NS
K = 80
NCHUNK = EPT // K
NLIN = N // 10


def _sc_edge_kernel(src_hbm, dst_hbm, asrc_hbm, adst_hbm, ht_hbm,
                    eatt_hbm, d_hbm, msg_hbm, deg_hbm,
                    pn1_v, pn2_v,
                    sr0, sr1, dsc0, dsc1, dx0, dx1, six0, six1,
                    ec0, ec1, at0, at1, ro0, ro1,
                    ones_v, zline_v,
                    iss0, iss1, isd0, isd1, ise0, ise1,
                    gs0, gs1, ss0, ss1, ew0, ew1, os0, os1,
                    d_sh, deg_sh, msg_sh):
    cid = lax.axis_index("c")
    sid = lax.axis_index("s")
    ebase = sid * EPT
    SR = (sr0, sr1)
    DS = (dsc0, dsc1)
    DX = (dx0, dx1)
    SI = (six0, six1)
    EC = (ec0, ec1)
    AT = (at0, at1)
    RO = (ro0, ro1)
    ISS = (iss0, iss1)
    ISD = (isd0, isd1)
    ISE = (ise0, ise1)
    GS = (gs0, gs1)
    SS = (ss0, ss1)
    EW = (ew0, ew1)
    OS = (os0, os1)

    def eoff(g):
        return jnp.minimum(ebase + g * K, E - K)

    @pl.loop(0, K // 16)
    def _ones(i):
        ones_v[pl.ds(i * 16, 16)] = jnp.ones((16,), jnp.float32)

    @pl.loop(0, 1024 // 16)
    def _zl(i):
        zline_v[pl.ds(i * 16, 16)] = jnp.zeros((16,), jnp.float32)

    for tt in range(2):
        t = cid * 2 + tt
        tN = t * N
        tE = t * E

        @pl.loop(0, K)
        def _zr(r):
            for q in range(D // 16):
                ro0[r, pl.ds(q * 16, 16)] = jnp.zeros((16,), jnp.float32)

        @pl.when(sid < 10)
        def _zacc():
            pltpu.sync_copy(zline_v.at[pl.ds(0, NLIN)],
                            d_sh.at[pl.ds(sid * NLIN, NLIN)])
            for z in range(NLIN // K):
                pltpu.sync_copy(ro0, msg_sh.at[pl.ds(sid * NLIN + z * K, K)])
            pltpu.sync_copy(ro0.at[pl.ds(0, NLIN - (NLIN // K) * K)],
                            msg_sh.at[pl.ds(sid * NLIN + (NLIN // K) * K,
                                            NLIN - (NLIN // K) * K)])

        if tt == 0:
            @pl.when(sid < 10)
            def _zdeg():
                pltpu.sync_copy(zline_v.at[pl.ds(0, NLIN)],
                                deg_sh.at[pl.ds(sid * NLIN, NLIN)])

        pltpu.sync_copy(asrc_hbm.at[pl.ds(tN, N)], pn1_v)
        pltpu.sync_copy(adst_hbm.at[pl.ds(tN, N)], pn2_v)

        plsc.subcore_barrier()

        def ig1(g, b):
            pltpu.async_copy(src_hbm.at[pl.ds(eoff(g), K)], SR[b], ISS[b])
            pltpu.async_copy(dst_hbm.at[pl.ds(eoff(g), K)], DS[b], ISD[b])

        def g1(g, b):
            pltpu.make_async_copy(
                src_hbm.at[pl.ds(eoff(g), K)], SR[b], ISS[b]).wait()
            pltpu.make_async_copy(
                dst_hbm.at[pl.ds(eoff(g), K)], DS[b], ISD[b]).wait()
            for i in range(K // 16):
                sl = pl.ds(i * 16, 16)
                s16 = SR[b][sl]
                d16 = DS[b][sl]
                DX[b][sl] = d16
                av = plsc.load_gather(pn1_v, [s16])
                bv = plsc.load_gather(pn2_v, [d16])
                sc = av + bv
                sc = jnp.where(sc >= 0.0, sc, 0.2 * sc)
                EC[b][sl] = jnp.exp(sc)
            pltpu.async_copy(EC[b], d_sh.at[DX[b]], SS[b], add=True)
            if tt == 0:
                pltpu.async_copy(ones_v, deg_sh.at[DX[b]], OS[b], add=True)
            pltpu.async_copy(
                EC[b], eatt_hbm.at[pl.ds(tE + ebase + g * K, K)], EW[b])
            ig1(g + 2, b)

        def ws1(g, b):
            pltpu.make_async_copy(EC[b], d_sh.at[DX[b]], SS[b]).wait()
            if tt == 0:
                pltpu.make_async_copy(ones_v, deg_sh.at[DX[b]], OS[b]).wait()
            pltpu.make_async_copy(
                EC[b], eatt_hbm.at[pl.ds(tE + ebase + g * K, K)],
                EW[b]).wait()

        ig1(0, 0)
        ig1(1, 1)
        g1(0, 0)
        g1(1, 1)

        @pl.loop(0, NCHUNK // 2 - 1)
        def _p1(p):
            g = 2 * p
            ws1(g, 0)
            g1(g + 2, 0)
            ws1(g + 1, 1)
            g1(g + 3, 1)

        ws1(NCHUNK - 2, 0)
        ws1(NCHUNK - 1, 1)
        for b in range(2):
            pltpu.make_async_copy(
                src_hbm.at[pl.ds(eoff(NCHUNK + b), K)], SR[b], ISS[b]).wait()
            pltpu.make_async_copy(
                dst_hbm.at[pl.ds(eoff(NCHUNK + b), K)], DS[b], ISD[b]).wait()

        plsc.subcore_barrier()

        pltpu.sync_copy(d_sh, pn2_v)

        @pl.when(sid < 10)
        def _dexp():
            pltpu.sync_copy(pn2_v.at[pl.ds(sid * NLIN, NLIN)],
                            d_hbm.at[pl.ds(tN + sid * NLIN, NLIN)])

        if tt == 0:
            @pl.when(jnp.logical_and(cid == 0, sid < 10))
            def _gexp():
                pltpu.sync_copy(deg_sh.at[pl.ds(sid * NLIN, NLIN)],
                                pn1_v.at[pl.ds(0, NLIN)])
                pltpu.sync_copy(pn1_v.at[pl.ds(0, NLIN)],
                                deg_hbm.at[pl.ds(sid * NLIN, NLIN)])

        def ig2(g, b):
            pltpu.async_copy(src_hbm.at[pl.ds(eoff(g), K)], SR[b], ISS[b])
            pltpu.async_copy(dst_hbm.at[pl.ds(eoff(g), K)], DS[b], ISD[b])
            pltpu.async_copy(
                eatt_hbm.at[pl.ds(jnp.minimum(tE + ebase + g * K,
                                              tE + E - K), K)],
                EC[b], ISE[b])

        def g2(g, b):
            pltpu.make_async_copy(
                src_hbm.at[pl.ds(eoff(g), K)], SR[b], ISS[b]).wait()
            pltpu.make_async_copy(
                dst_hbm.at[pl.ds(eoff(g), K)], DS[b], ISD[b]).wait()
            pltpu.make_async_copy(
                eatt_hbm.at[pl.ds(tE + ebase + g * K, K)], EC[b],
                ISE[b]).wait()
            for i in range(K // 16):
                sl = pl.ds(i * 16, 16)
                s16 = SR[b][sl]
                d16 = DS[b][sl]
                SI[b][sl] = s16 + tN
                DX[b][sl] = d16
                dd = plsc.load_gather(pn2_v, [d16])
                AT[b][sl] = EC[b][sl] / (dd + 1e-9)
            pltpu.async_copy(ht_hbm.at[SI[b]], RO[b], GS[b])
            ig2(g + 2, b)

        def p2(g, b):
            pltpu.make_async_copy(ht_hbm.at[SI[b]], RO[b], GS[b]).wait()

            @pl.loop(0, K, unroll=8)
            def _scale(r):
                ab = plsc.load_gather(AT[b], [jnp.full((16,), r, jnp.int32)])
                for q in range(D // 16):
                    RO[b][r, pl.ds(q * 16, 16)] = (
                        RO[b][r, pl.ds(q * 16, 16)] * ab)

            pltpu.async_copy(RO[b], msg_sh.at[DX[b]], SS[b], add=True)
            pltpu.async_copy(
                AT[b], eatt_hbm.at[pl.ds(tE + ebase + g * K, K)], EW[b])

        def ws2(g, b):
            pltpu.make_async_copy(RO[b], msg_sh.at[DX[b]], SS[b]).wait()
            pltpu.make_async_copy(
                AT[b], eatt_hbm.at[pl.ds(tE + ebase + g * K, K)],
                EW[b]).wait()

        ig2(0, 0)
        ig2(1, 1)
        g2(0, 0)
        g2(1, 1)

        @pl.loop(0, NCHUNK // 2 - 1)
        def _p2(p):
            g = 2 * p
            p2(g, 0)
            p2(g + 1, 1)
            ws2(g, 0)
            g2(g + 2, 0)
            ws2(g + 1, 1)
            g2(g + 3, 1)

        p2(NCHUNK - 2, 0)
        p2(NCHUNK - 1, 1)
        ws2(NCHUNK - 2, 0)
        ws2(NCHUNK - 1, 1)
        for b in range(2):
            pltpu.make_async_copy(
                src_hbm.at[pl.ds(eoff(NCHUNK + b), K)], SR[b], ISS[b]).wait()
            pltpu.make_async_copy(
                dst_hbm.at[pl.ds(eoff(NCHUNK + b), K)], DS[b], ISD[b]).wait()
            pltpu.make_async_copy(
                eatt_hbm.at[pl.ds(jnp.minimum(tE + ebase + (NCHUNK + b) * K,
                                              tE + E - K), K)],
                EC[b], ISE[b]).wait()

        plsc.subcore_barrier()

        @pl.when(sid < 10)
        def _mexp():
            pltpu.sync_copy(msg_sh.at[pl.ds(sid * NLIN, NLIN)],
                            msg_hbm.at[pl.ds(tN + sid * NLIN, NLIN)])

        plsc.subcore_barrier()


def _run_sc_edge(asrc, adst, src, dst, ht):
    f32 = jnp.float32
    i32 = jnp.int32
    kern = pl.kernel(
        _sc_edge_kernel,
        out_type=[
            jax.ShapeDtypeStruct((T * E,), f32),
            jax.ShapeDtypeStruct((T * N,), f32),
            jax.ShapeDtypeStruct((T * N, D), f32),
            jax.ShapeDtypeStruct((N,), f32),
        ],
        mesh=plsc.VectorSubcoreMesh(core_axis_name="c", subcore_axis_name="s"),
        scratch_types=(
            [pltpu.VMEM((N,), f32) for _ in range(2)]
            + [pltpu.VMEM((K,), i32) for _ in range(8)]
            + [pltpu.VMEM((K,), f32) for _ in range(4)]
            + [pltpu.VMEM((K, D), f32) for _ in range(2)]
            + [pltpu.VMEM((K,), f32), pltpu.VMEM((1024,), f32)]
            + [pltpu.SemaphoreType.DMA for _ in range(14)]
            + [pltpu.VMEM_SHARED((N,), f32),
               pltpu.VMEM_SHARED((N,), f32),
               pltpu.VMEM_SHARED((N, D), f32)]
        ),
        compiler_params=pltpu.CompilerParams(needs_layout_passes=False),
    )
    return kern(src, dst, asrc, adst, ht)


RB = 1000
NB = N // RB


def _mm_kernel(x_ref, w_ref, o_ref):
    o_ref[...] = jnp.dot(x_ref[...], w_ref[...],
                         preferred_element_type=jnp.float32)


def _run_matmul(x_flat, w_aug):
    M = T * N
    return pl.pallas_call(
        _mm_kernel,
        grid=(M // RB,),
        in_specs=[
            pl.BlockSpec((RB, D), lambda i: (i, 0)),
            pl.BlockSpec((D, 2 * D), lambda i: (0, 0)),
        ],
        out_specs=pl.BlockSpec((RB, 2 * D), lambda i: (i, 0)),
        out_shape=jax.ShapeDtypeStruct((M, 2 * D), jnp.float32),
    )(x_flat, w_aug)


def _readout_kernel(msg_ref, n2g_ref, d_ref, deg_ref, g_ref, side_ref):
    j = pl.program_id(1)

    @pl.when(j == 0)
    def _init():
        g_ref[...] = jnp.zeros((1, B, D), jnp.float32)
        side_ref[...] = jnp.zeros((1, B, D), jnp.float32)

    nh = jnp.maximum(msg_ref[0], 0.0)
    n2g = n2g_ref[0, 0]
    gid = lax.broadcasted_iota(jnp.int32, (B, RB), 0)
    s_f = (jnp.broadcast_to(n2g[None, :], (B, RB)) == gid).astype(jnp.float32)

    dv = d_ref[0, 0]
    attsum = dv / (dv + 1e-9)
    degv = deg_ref[0, 0]
    lane = lax.broadcasted_iota(jnp.int32, (RB, D), 1)
    side = (jnp.where(lane == 0, attsum[:, None], 0.0)
            + jnp.where(lane == 1, degv[:, None], 0.0))

    g_ref[0] += jnp.dot(s_f, nh, preferred_element_type=jnp.float32)
    side_ref[0] += jnp.dot(s_f, side, preferred_element_type=jnp.float32)


def _run_readout(msg, n2g, d_arr, deg):
    msg_r = msg.reshape(T * NB, RB, D)
    n2g_r = n2g.reshape(NB, 1, RB)
    d_r = d_arr.reshape(T * NB, 1, RB)
    deg_r = deg.reshape(NB, 1, RB)
    return pl.pallas_call(
        _readout_kernel,
        grid=(T, NB),
        in_specs=[
            pl.BlockSpec((1, RB, D), lambda t, j: (t * NB + j, 0, 0)),
            pl.BlockSpec((1, 1, RB), lambda t, j: (j, 0, 0)),
            pl.BlockSpec((1, 1, RB), lambda t, j: (t * NB + j, 0, 0)),
            pl.BlockSpec((1, 1, RB), lambda t, j: (j, 0, 0)),
        ],
        out_specs=[
            pl.BlockSpec((1, B, D), lambda t, j: (t, 0, 0)),
            pl.BlockSpec((1, B, D), lambda t, j: (t, 0, 0)),
        ],
        out_shape=[
            jax.ShapeDtypeStruct((T, B, D), jnp.float32),
            jax.ShapeDtypeStruct((T, B, D), jnp.float32),
        ],
        compiler_params=pltpu.CompilerParams(
            dimension_semantics=("arbitrary", "arbitrary")),
    )(msg_r, n2g_r, d_r, deg_r)


def _lstm_kernel(g_ref, side_ref, wih_ref, whh_ref, wout_ref, bg_ref,
                 bo_ref, ps_ref, ra_ref):
    h = jnp.zeros((B, D), jnp.float32)
    c = jnp.zeros((B, D), jnp.float32)
    lane = lax.broadcasted_iota(jnp.int32, (B, D), 1)
    for t in range(T):
        xg = g_ref[t] + h
        gates = (jnp.dot(xg, wih_ref[...], preferred_element_type=jnp.float32)
                 + jnp.dot(h, whh_ref[...], preferred_element_type=jnp.float32)
                 + bg_ref[...])
        i_g = gates[:, 0 * D:1 * D]
        f_g = gates[:, 1 * D:2 * D]
        g_g = gates[:, 2 * D:3 * D]
        o_g = gates[:, 3 * D:4 * D]
        c = jax.nn.sigmoid(f_g) * c + jax.nn.sigmoid(i_g) * jnp.tanh(g_g)
        h = jax.nn.sigmoid(o_g) * jnp.tanh(c)
        logits = jnp.dot(h, wout_ref[...],
                         preferred_element_type=jnp.float32) + bo_ref[...]
        ml = jnp.where(lane < C, logits, -1e30)
        m = jnp.max(ml, axis=1, keepdims=True)
        e = jnp.exp(ml - m)
        ps_ref[t] = e / jnp.sum(e, axis=1, keepdims=True)

        sd = side_ref[t]
        ra = sd[:, 0:1] / (sd[:, 1:2] + 1e-9)
        ra_ref[t] = jnp.broadcast_to(ra, (B, D))


def _run_lstm(g_all, side_all, wih_t, whh_t, wout_t, bg, bo):
    return pl.pallas_call(
        _lstm_kernel,
        out_shape=[
            jax.ShapeDtypeStruct((T, B, D), jnp.float32),
            jax.ShapeDtypeStruct((T, B, D), jnp.float32),
        ],
    )(g_all, side_all, wih_t, whh_t, wout_t, bg, bo)


def kernel(x, edge_index, node2graph, mask, W, a_src, a_dst,
           W_ih, W_hh, b_ih, b_hh, W_out, b_out):
    f32 = jnp.float32
    x_flat = x.reshape(T * N, D).astype(f32)

    w_s = W @ a_src
    w_d = W @ a_dst
    w_aug = jnp.concatenate(
        [W, w_s[:, None], w_d[:, None], jnp.zeros((D, D - 2), f32)], axis=1)

    ht_aug = _run_matmul(x_flat, w_aug)
    ht = ht_aug[:, :D] + 0.0
    asrc = ht_aug[:, D] + 0.0
    adst = ht_aug[:, D + 1] + 0.0

    src = edge_index[0].astype(jnp.int32)
    dst = edge_index[1].astype(jnp.int32)

    eatt, d_arr, msg, deg = _run_sc_edge(asrc, adst, src, dst, ht)

    g_all, side_all = _run_readout(msg, node2graph.astype(jnp.int32),
                                   d_arr, deg)

    bg = (b_ih + b_hh).reshape(1, 4 * D).astype(f32)
    bo = jnp.concatenate([b_out, jnp.zeros((D - C,), f32)]).reshape(1, D)
    wout_t = jnp.concatenate([W_out.T, jnp.zeros((D, D - C), f32)], axis=1)

    ps_pad, ra_pad = _run_lstm(g_all, side_all,
                               W_ih.T.astype(f32), W_hh.T.astype(f32),
                               wout_t.astype(f32), bg, bo)

    ps = ps_pad[:, :, :C]
    rel_atts = ra_pad[:, :, 0]
    edge_atts = eatt.reshape(T, E)
    return (ps, rel_atts, edge_atts)


# speedup vs baseline: 1.1440x; 1.0979x over previous
import jax
import jax.numpy as jnp
from jax import lax
from jax.experimental import pallas as pl
from jax.experimental.pallas import tpu as pltpu
from jax.experimental.pallas import tpu_sc as plsc

T, N, E, B, D, C = 4, 10000, 320000, 64, 128, 10
NS = 16
EPT = E // NS
K = 80
NCHUNK = EPT // K
NLIN = N // 10


def _sc_edge_kernel(src_hbm, dst_hbm, asrc_hbm, adst_hbm, ht_hbm,
                    eatt_hbm, d_hbm, msg_hbm, deg_hbm,
                    pn1_v, pn2_v,
                    sr0, sr1, dsc0, dsc1, dx0, dx1, six0, six1,
                    ec0, ec1, at0, at1, ro0, ro1,
                    ones_v, zline_v,
                    iss0, iss1, isd0, isd1, ise0, ise1,
                    gs0, gs1, ss0, ss1, ew0, ew1, os0, os1,
                    d_sh, deg_sh, msg_sh):
    cid = lax.axis_index("c")
    sid = lax.axis_index("s")
    ebase = sid * EPT
    SR = (sr0, sr1)
    DS = (dsc0, dsc1)
    DX = (dx0, dx1)
    SI = (six0, six1)
    EC = (ec0, ec1)
    AT = (at0, at1)
    RO = (ro0, ro1)
    ISS = (iss0, iss1)
    ISD = (isd0, isd1)
    ISE = (ise0, ise1)
    GS = (gs0, gs1)
    SS = (ss0, ss1)
    EW = (ew0, ew1)
    OS = (os0, os1)

    def eoff(g):
        return jnp.minimum(ebase + g * K, E - K)

    @pl.loop(0, K // 16)
    def _ones(i):
        ones_v[pl.ds(i * 16, 16)] = jnp.ones((16,), jnp.float32)

    @pl.loop(0, 1024 // 16)
    def _zl(i):
        zline_v[pl.ds(i * 16, 16)] = jnp.zeros((16,), jnp.float32)

    for tt in range(2):
        t = cid * 2 + tt
        tN = t * N
        tE = t * E

        @pl.loop(0, K)
        def _zr(r):
            for q in range(D // 16):
                ro0[r, pl.ds(q * 16, 16)] = jnp.zeros((16,), jnp.float32)

        @pl.when(sid < 10)
        def _zacc():
            pltpu.sync_copy(zline_v.at[pl.ds(0, NLIN)],
                            d_sh.at[pl.ds(sid * NLIN, NLIN)])
            for z in range(NLIN // K):
                pltpu.sync_copy(ro0, msg_sh.at[pl.ds(sid * NLIN + z * K, K)])
            pltpu.sync_copy(ro0.at[pl.ds(0, NLIN - (NLIN // K) * K)],
                            msg_sh.at[pl.ds(sid * NLIN + (NLIN // K) * K,
                                            NLIN - (NLIN // K) * K)])

        if tt == 0:
            @pl.when(sid < 10)
            def _zdeg():
                pltpu.sync_copy(zline_v.at[pl.ds(0, NLIN)],
                                deg_sh.at[pl.ds(sid * NLIN, NLIN)])

        pltpu.sync_copy(asrc_hbm.at[pl.ds(tN, N)], pn1_v)
        pltpu.sync_copy(adst_hbm.at[pl.ds(tN, N)], pn2_v)

        plsc.subcore_barrier()

        def ig1(g, b):
            pltpu.async_copy(src_hbm.at[pl.ds(eoff(g), K)], SR[b], ISS[b])
            pltpu.async_copy(dst_hbm.at[pl.ds(eoff(g), K)], DS[b], ISD[b])

        def g1(g, b):
            pltpu.make_async_copy(
                src_hbm.at[pl.ds(eoff(g), K)], SR[b], ISS[b]).wait()
            pltpu.make_async_copy(
                dst_hbm.at[pl.ds(eoff(g), K)], DS[b], ISD[b]).wait()
            for i in range(K // 16):
                sl = pl.ds(i * 16, 16)
                s16 = SR[b][sl]
                d16 = DS[b][sl]
                DX[b][sl] = d16
                av = plsc.load_gather(pn1_v, [s16])
                bv = plsc.load_gather(pn2_v, [d16])
                sc = av + bv
                sc = jnp.where(sc >= 0.0, sc, 0.2 * sc)
                EC[b][sl] = jnp.exp(sc)
            pltpu.async_copy(EC[b], d_sh.at[DX[b]], SS[b], add=True)
            if tt == 0:
                pltpu.async_copy(ones_v, deg_sh.at[DX[b]], OS[b], add=True)
            pltpu.async_copy(
                EC[b], eatt_hbm.at[pl.ds(tE + ebase + g * K, K)], EW[b])
            ig1(g + 2, b)

        def ws1(g, b):
            pltpu.make_async_copy(EC[b], d_sh.at[DX[b]], SS[b]).wait()
            if tt == 0:
                pltpu.make_async_copy(ones_v, deg_sh.at[DX[b]], OS[b]).wait()
            pltpu.make_async_copy(
                EC[b], eatt_hbm.at[pl.ds(tE + ebase + g * K, K)],
                EW[b]).wait()

        ig1(0, 0)
        ig1(1, 1)
        g1(0, 0)
        g1(1, 1)

        @pl.loop(0, NCHUNK // 2 - 1)
        def _p1(p):
            g = 2 * p
            ws1(g, 0)
            g1(g + 2, 0)
            ws1(g + 1, 1)
            g1(g + 3, 1)

        ws1(NCHUNK - 2, 0)
        ws1(NCHUNK - 1, 1)
        for b in range(2):
            pltpu.make_async_copy(
                src_hbm.at[pl.ds(eoff(NCHUNK + b), K)], SR[b], ISS[b]).wait()
            pltpu.make_async_copy(
                dst_hbm.at[pl.ds(eoff(NCHUNK + b), K)], DS[b], ISD[b]).wait()

        plsc.subcore_barrier()

        pltpu.sync_copy(d_sh, pn2_v)

        @pl.when(sid < 10)
        def _dexp():
            pltpu.sync_copy(pn2_v.at[pl.ds(sid * NLIN, NLIN)],
                            d_hbm.at[pl.ds(tN + sid * NLIN, NLIN)])

        if tt == 0:
            @pl.when(jnp.logical_and(cid == 0, sid < 10))
            def _gexp():
                pltpu.sync_copy(deg_sh.at[pl.ds(sid * NLIN, NLIN)],
                                pn1_v.at[pl.ds(0, NLIN)])
                pltpu.sync_copy(pn1_v.at[pl.ds(0, NLIN)],
                                deg_hbm.at[pl.ds(sid * NLIN, NLIN)])

        def ig2(g, b):
            pltpu.async_copy(src_hbm.at[pl.ds(eoff(g), K)], SR[b], ISS[b])
            pltpu.async_copy(dst_hbm.at[pl.ds(eoff(g), K)], DS[b], ISD[b])
            pltpu.async_copy(
                eatt_hbm.at[pl.ds(jnp.minimum(tE + ebase + g * K,
                                              tE + E - K), K)],
                EC[b], ISE[b])

        def g2(g, b):
            pltpu.make_async_copy(
                src_hbm.at[pl.ds(eoff(g), K)], SR[b], ISS[b]).wait()
            pltpu.make_async_copy(
                dst_hbm.at[pl.ds(eoff(g), K)], DS[b], ISD[b]).wait()
            pltpu.make_async_copy(
                eatt_hbm.at[pl.ds(tE + ebase + g * K, K)], EC[b],
                ISE[b]).wait()
            for i in range(K // 16):
                sl = pl.ds(i * 16, 16)
                s16 = SR[b][sl]
                d16 = DS[b][sl]
                SI[b][sl] = s16 + tN
                DX[b][sl] = d16
                dd = plsc.load_gather(pn2_v, [d16])
                AT[b][sl] = EC[b][sl] / (dd + 1e-9)
            pltpu.async_copy(ht_hbm.at[SI[b]], RO[b], GS[b])
            ig2(g + 2, b)

        def p2(g, b):
            pltpu.make_async_copy(ht_hbm.at[SI[b]], RO[b], GS[b]).wait()

            @plsc.parallel_loop(0, K, unroll=8)
            def _scale(r):
                ab = plsc.load_gather(AT[b], [jnp.full((16,), r, jnp.int32)])
                for q in range(D // 16):
                    RO[b][r, pl.ds(q * 16, 16)] = (
                        RO[b][r, pl.ds(q * 16, 16)] * ab)

            pltpu.async_copy(RO[b], msg_sh.at[DX[b]], SS[b], add=True)
            pltpu.async_copy(
                AT[b], eatt_hbm.at[pl.ds(tE + ebase + g * K, K)], EW[b])

        def ws2(g, b):
            pltpu.make_async_copy(RO[b], msg_sh.at[DX[b]], SS[b]).wait()
            pltpu.make_async_copy(
                AT[b], eatt_hbm.at[pl.ds(tE + ebase + g * K, K)],
                EW[b]).wait()

        ig2(0, 0)
        ig2(1, 1)
        g2(0, 0)
        g2(1, 1)

        @pl.loop(0, NCHUNK // 2 - 1)
        def _p2(p):
            g = 2 * p
            p2(g, 0)
            p2(g + 1, 1)
            ws2(g, 0)
            g2(g + 2, 0)
            ws2(g + 1, 1)
            g2(g + 3, 1)

        p2(NCHUNK - 2, 0)
        p2(NCHUNK - 1, 1)
        ws2(NCHUNK - 2, 0)
        ws2(NCHUNK - 1, 1)
        for b in range(2):
            pltpu.make_async_copy(
                src_hbm.at[pl.ds(eoff(NCHUNK + b), K)], SR[b], ISS[b]).wait()
            pltpu.make_async_copy(
                dst_hbm.at[pl.ds(eoff(NCHUNK + b), K)], DS[b], ISD[b]).wait()
            pltpu.make_async_copy(
                eatt_hbm.at[pl.ds(jnp.minimum(tE + ebase + (NCHUNK + b) * K,
                                              tE + E - K), K)],
                EC[b], ISE[b]).wait()

        plsc.subcore_barrier()

        @pl.when(sid < 10)
        def _mexp():
            pltpu.sync_copy(msg_sh.at[pl.ds(sid * NLIN, NLIN)],
                            msg_hbm.at[pl.ds(tN + sid * NLIN, NLIN)])

        plsc.subcore_barrier()


def _run_sc_edge(asrc, adst, src, dst, ht):
    f32 = jnp.float32
    i32 = jnp.int32
    kern = pl.kernel(
        _sc_edge_kernel,
        out_type=[
            jax.ShapeDtypeStruct((T * E,), f32),
            jax.ShapeDtypeStruct((T * N,), f32),
            jax.ShapeDtypeStruct((T * N, D), f32),
            jax.ShapeDtypeStruct((N,), f32),
        ],
        mesh=plsc.VectorSubcoreMesh(core_axis_name="c", subcore_axis_name="s"),
        scratch_types=(
            [pltpu.VMEM((N,), f32) for _ in range(2)]
            + [pltpu.VMEM((K,), i32) for _ in range(8)]
            + [pltpu.VMEM((K,), f32) for _ in range(4)]
            + [pltpu.VMEM((K, D), f32) for _ in range(2)]
            + [pltpu.VMEM((K,), f32), pltpu.VMEM((1024,), f32)]
            + [pltpu.SemaphoreType.DMA for _ in range(14)]
            + [pltpu.VMEM_SHARED((N,), f32),
               pltpu.VMEM_SHARED((N,), f32),
               pltpu.VMEM_SHARED((N, D), f32)]
        ),
        compiler_params=pltpu.CompilerParams(needs_layout_passes=False),
    )
    return kern(src, dst, asrc, adst, ht)


RB = 1000
NB = N // RB


def _mm_kernel(x_ref, w_ref, o_ref):
    o_ref[...] = jnp.dot(x_ref[...], w_ref[...],
                         preferred_element_type=jnp.float32)


def _run_matmul(x_flat, w_aug):
    M = T * N
    return pl.pallas_call(
        _mm_kernel,
        grid=(M // RB,),
        in_specs=[
            pl.BlockSpec((RB, D), lambda i: (i, 0)),
            pl.BlockSpec((D, 2 * D), lambda i: (0, 0)),
        ],
        out_specs=pl.BlockSpec((RB, 2 * D), lambda i: (i, 0)),
        out_shape=jax.ShapeDtypeStruct((M, 2 * D), jnp.float32),
    )(x_flat, w_aug)


def _readout_kernel(msg_ref, n2g_ref, d_ref, deg_ref, g_ref, side_ref):
    j = pl.program_id(1)

    @pl.when(j == 0)
    def _init():
        g_ref[...] = jnp.zeros((1, B, D), jnp.float32)
        side_ref[...] = jnp.zeros((1, B, D), jnp.float32)

    nh = jnp.maximum(msg_ref[0], 0.0)
    n2g = n2g_ref[0, 0]
    gid = lax.broadcasted_iota(jnp.int32, (B, RB), 0)
    s_f = (jnp.broadcast_to(n2g[None, :], (B, RB)) == gid).astype(jnp.float32)

    dv = d_ref[0, 0]
    attsum = dv / (dv + 1e-9)
    degv = deg_ref[0, 0]
    lane = lax.broadcasted_iota(jnp.int32, (RB, D), 1)
    side = (jnp.where(lane == 0, attsum[:, None], 0.0)
            + jnp.where(lane == 1, degv[:, None], 0.0))

    g_ref[0] += jnp.dot(s_f, nh, preferred_element_type=jnp.float32)
    side_ref[0] += jnp.dot(s_f, side, preferred_element_type=jnp.float32)


def _run_readout(msg, n2g, d_arr, deg):
    msg_r = msg.reshape(T * NB, RB, D)
    n2g_r = n2g.reshape(NB, 1, RB)
    d_r = d_arr.reshape(T * NB, 1, RB)
    deg_r = deg.reshape(NB, 1, RB)
    return pl.pallas_call(
        _readout_kernel,
        grid=(T, NB),
        in_specs=[
            pl.BlockSpec((1, RB, D), lambda t, j: (t * NB + j, 0, 0)),
            pl.BlockSpec((1, 1, RB), lambda t, j: (j, 0, 0)),
            pl.BlockSpec((1, 1, RB), lambda t, j: (t * NB + j, 0, 0)),
            pl.BlockSpec((1, 1, RB), lambda t, j: (j, 0, 0)),
        ],
        out_specs=[
            pl.BlockSpec((1, B, D), lambda t, j: (t, 0, 0)),
            pl.BlockSpec((1, B, D), lambda t, j: (t, 0, 0)),
        ],
        out_shape=[
            jax.ShapeDtypeStruct((T, B, D), jnp.float32),
            jax.ShapeDtypeStruct((T, B, D), jnp.float32),
        ],
        compiler_params=pltpu.CompilerParams(
            dimension_semantics=("arbitrary", "arbitrary")),
    )(msg_r, n2g_r, d_r, deg_r)


def _lstm_kernel(g_ref, side_ref, wih_ref, whh_ref, wout_ref, bg_ref,
                 bo_ref, ps_ref, ra_ref):
    h = jnp.zeros((B, D), jnp.float32)
    c = jnp.zeros((B, D), jnp.float32)
    lane = lax.broadcasted_iota(jnp.int32, (B, D), 1)
    for t in range(T):
        xg = g_ref[t] + h
        gates = (jnp.dot(xg, wih_ref[...], preferred_element_type=jnp.float32)
                 + jnp.dot(h, whh_ref[...], preferred_element_type=jnp.float32)
                 + bg_ref[...])
        i_g = gates[:, 0 * D:1 * D]
        f_g = gates[:, 1 * D:2 * D]
        g_g = gates[:, 2 * D:3 * D]
        o_g = gates[:, 3 * D:4 * D]
        c = jax.nn.sigmoid(f_g) * c + jax.nn.sigmoid(i_g) * jnp.tanh(g_g)
        h = jax.nn.sigmoid(o_g) * jnp.tanh(c)
        logits = jnp.dot(h, wout_ref[...],
                         preferred_element_type=jnp.float32) + bo_ref[...]
        ml = jnp.where(lane < C, logits, -1e30)
        m = jnp.max(ml, axis=1, keepdims=True)
        e = jnp.exp(ml - m)
        ps_ref[t] = e / jnp.sum(e, axis=1, keepdims=True)

        sd = side_ref[t]
        ra = sd[:, 0:1] / (sd[:, 1:2] + 1e-9)
        ra_ref[t] = jnp.broadcast_to(ra, (B, D))


def _run_lstm(g_all, side_all, wih_t, whh_t, wout_t, bg, bo):
    return pl.pallas_call(
        _lstm_kernel,
        out_shape=[
            jax.ShapeDtypeStruct((T, B, D), jnp.float32),
            jax.ShapeDtypeStruct((T, B, D), jnp.float32),
        ],
    )(g_all, side_all, wih_t, whh_t, wout_t, bg, bo)


def kernel(x, edge_index, node2graph, mask, W, a_src, a_dst,
           W_ih, W_hh, b_ih, b_hh, W_out, b_out):
    f32 = jnp.float32
    x_flat = x.reshape(T * N, D).astype(f32)

    w_s = W @ a_src
    w_d = W @ a_dst
    w_aug = jnp.concatenate(
        [W, w_s[:, None], w_d[:, None], jnp.zeros((D, D - 2), f32)], axis=1)

    ht_aug = _run_matmul(x_flat, w_aug)
    ht = ht_aug[:, :D] + 0.0
    asrc = ht_aug[:, D] + 0.0
    adst = ht_aug[:, D + 1] + 0.0

    src = edge_index[0].astype(jnp.int32)
    dst = edge_index[1].astype(jnp.int32)

    eatt, d_arr, msg, deg = _run_sc_edge(asrc, adst, src, dst, ht)

    g_all, side_all = _run_readout(msg, node2graph.astype(jnp.int32),
                                   d_arr, deg)

    bg = (b_ih + b_hh).reshape(1, 4 * D).astype(f32)
    bo = jnp.concatenate([b_out, jnp.zeros((D - C,), f32)]).reshape(1, D)
    wout_t = jnp.concatenate([W_out.T, jnp.zeros((D, D - C), f32)], axis=1)

    ps_pad, ra_pad = _run_lstm(g_all, side_all,
                               W_ih.T.astype(f32), W_hh.T.astype(f32),
                               wout_t.astype(f32), bg, bo)

    ps = ps_pad[:, :, :C]
    rel_atts = ra_pad[:, :, 0]
    edge_atts = eatt.reshape(T, E)
    return (ps, rel_atts, edge_atts)
